# baseline (speedup 1.0000x reference)
import jax
import jax.numpy as jnp
from jax import lax
from jax.experimental import pallas as pl
from jax.experimental.pallas import tpu as pltpu

_IMG = 640.0
_MAX_DET = 128
_N = 16800
_ROWS = 132


def _topk_body(s_ref, sig_ref, idx_ref, ndet_ref, S):
    B = 16
    S[...] = s_ref[...]
    cio = lax.broadcasted_iota(jnp.int32, (B, _ROWS), 1)
    li = lax.broadcasted_iota(jnp.int32, (B, 128), 1)
    M0 = jnp.max(S[...], axis=2)

    def step(j, carry):
        M, sv, iv = carry
        gm = jnp.max(M, axis=1, keepdims=True)
        rvec = jnp.min(jnp.where(M == gm, cio, jnp.int32(100000)),
                       axis=1, keepdims=True)
        rbs = [rvec[b, 0] for b in range(B)]
        R = jnp.concatenate(
            [S[b, pl.ds(rbs[b], 1), :] for b in range(B)], axis=0)
        cvec = jnp.min(jnp.where(R == gm, li, jnp.int32(100000)),
                       axis=1, keepdims=True)
        sv = jnp.where(li == j, gm, sv)
        iv = jnp.where(li == j, rvec * 128 + cvec, iv)
        R2 = jnp.where(li == cvec, -jnp.inf, R)
        for b in range(B):
            S[b, pl.ds(rbs[b], 1), :] = R2[b:b + 1, :]
        nm = jnp.max(R2, axis=1, keepdims=True)
        M = jnp.where(cio == rvec, nm, M)
        return M, sv, iv

    _, sv, iv = lax.fori_loop(
        0, _MAX_DET, step,
        (M0, jnp.full((B, 128), -jnp.inf, jnp.float32),
         jnp.zeros((B, 128), jnp.int32)))
    sig = 1.0 / (1.0 + jnp.exp(-sv))
    sig_ref[...] = sig
    idx_ref[...] = iv
    ndet_ref[...] = jnp.sum(jnp.where(sig > 0.5, 1, 0), axis=1,
                            keepdims=True).astype(jnp.int32)


def _i2f(x):
    return lax.bitcast_convert_type(x + 0x4B000000, jnp.float32) - 8388608.0

def _dec_body(idxf, b8f, b16f, b32f, l8f, l16f, l32f,
              bxo, lmo,
              tibuf, E8, E16, E32, W8, W16, W32, ob, ol,
              sem0, sem1, sem2, sem3, sem4, sem5):
    wid = lax.axis_index("s") * 2 + lax.axis_index("c")

    @pl.when(wid < 16)
    def _():
        b = wid
        pltpu.sync_copy(idxf.at[pl.ds(b * 128, 128)], tibuf)

        def anchors(ti):
            def lvl(off, f, st):
                a = jnp.maximum((ti - off) >> 1, 0)
                m = {80: 52429, 40: 104858, 20: 209716}[f]
                y = (a * m) >> 22
                x = a - y * f
                return ((_i2f(x) + 0.5) * st, (_i2f(y) + 0.5) * st)
            cx8, cy8 = lvl(0, 80, 8.0)
            cx16, cy16 = lvl(12800, 40, 16.0)
            cx32, cy32 = lvl(16000, 20, 32.0)
            m16 = ti >= 12800
            m32 = ti >= 16000
            cx = jnp.where(m32, cx32, jnp.where(m16, cx16, cx8))
            cy = jnp.where(m32, cy32, jnp.where(m16, cy16, cy8))
            st = jnp.where(m32, 32.0, jnp.where(m16, 16.0, 8.0))
            return cx, cy, st, m16, m32

        for comp in range(4):
            def fb(i, x, comp=comp):
                ti = tibuf[pl.ds(i * 16, 16)]
                E8[comp, pl.ds(i * 16, 16)] = (
                    (b * 12800 + jnp.clip(ti, 0, 12799)) * 4 + comp)
                E16[comp, pl.ds(i * 16, 16)] = (
                    (b * 3200 + jnp.clip(ti - 12800, 0, 3199)) * 4 + comp)
                E32[comp, pl.ds(i * 16, 16)] = (
                    (b * 800 + jnp.clip(ti - 16000, 0, 799)) * 4 + comp)
                return x
            lax.fori_loop(0, 8, fb, jnp.int32(0))
        for t in range(10):
            def fl(i, x, t=t):
                ti = tibuf[pl.ds(i * 16, 16)]
                E8[4 + t, pl.ds(i * 16, 16)] = (
                    (b * 12800 + jnp.clip(ti, 0, 12799)) * 10 + t)
                E16[4 + t, pl.ds(i * 16, 16)] = (
                    (b * 3200 + jnp.clip(ti - 12800, 0, 3199)) * 10 + t)
                E32[4 + t, pl.ds(i * 16, 16)] = (
                    (b * 800 + jnp.clip(ti - 16000, 0, 799)) * 10 + t)
                return x
            lax.fori_loop(0, 8, fl, jnp.int32(0))

        sems = [sem0, sem1, sem2, sem3, sem4, sem5]
        cps = []
        for k in range(14):
            t8 = b8f if k < 4 else l8f
            t16 = b16f if k < 4 else l16f
            t32 = b32f if k < 4 else l32f
            cps.append(pltpu.async_copy(t8.at[E8.at[k]], W8.at[k],
                                        sems[(3 * k) % 6]))
            cps.append(pltpu.async_copy(t16.at[E16.at[k]], W16.at[k],
                                        sems[(3 * k + 1) % 6]))
            cps.append(pltpu.async_copy(t32.at[E32.at[k]], W32.at[k],
                                        sems[(3 * k + 2) % 6]))
        for cp in cps:
            cp.wait()

        for comp in range(4):
            sgn = 1.0 if comp >= 2 else -1.0

            def db(i, x, comp=comp, sgn=sgn):
                ti = tibuf[pl.ds(i * 16, 16)]
                cx, cy, st, m16, m32 = anchors(ti)
                v = jnp.where(m32, W32[comp, pl.ds(i * 16, 16)],
                              jnp.where(m16, W16[comp, pl.ds(i * 16, 16)],
                                        W8[comp, pl.ds(i * 16, 16)]))
                ctr = cy if comp % 2 == 1 else cx
                ob[pl.ds(comp * 128 + i * 16, 16)] = (
                    (ctr + sgn * v * st) / _IMG)
                return x
            lax.fori_loop(0, 8, db, jnp.int32(0))
        pltpu.sync_copy(ob, bxo.at[pl.ds(b * 512, 512)])

        for t in range(10):
            def dl(i, x, t=t):
                ti = tibuf[pl.ds(i * 16, 16)]
                cx, cy, st, m16, m32 = anchors(ti)
                v = jnp.where(m32, W32[4 + t, pl.ds(i * 16, 16)],
                              jnp.where(m16, W16[4 + t, pl.ds(i * 16, 16)],
                                        W8[4 + t, pl.ds(i * 16, 16)]))
                ctr = cy if t % 2 == 1 else cx
                ol[pl.ds(t * 128 + i * 16, 16)] = (v * st + ctr) / _IMG
                return x
            lax.fori_loop(0, 8, dl, jnp.int32(0))
        pltpu.sync_copy(ol, lmo.at[pl.ds(b * 1280, 1280)])


def _sc_decode(idx, boxes_8, boxes_16, boxes_32,
               landmarks_8, landmarks_16, landmarks_32):
    from jax.experimental.pallas import tpu_sc as plsc
    B = idx.shape[0]
    f32, i32 = jnp.float32, jnp.int32
    mesh = plsc.VectorSubcoreMesh(core_axis_name="c", subcore_axis_name="s")
    run = pl.kernel(
        _dec_body,
        out_type=[
            jax.ShapeDtypeStruct((B * 512,), f32),
            jax.ShapeDtypeStruct((B * 1280,), f32),
        ],
        mesh=mesh,
        scratch_types=[
            pltpu.VMEM((128,), i32),
            pltpu.VMEM((14, 128), i32),
            pltpu.VMEM((14, 128), i32),
            pltpu.VMEM((14, 128), i32),
            pltpu.VMEM((14, 128), f32),
            pltpu.VMEM((14, 128), f32),
            pltpu.VMEM((14, 128), f32),
            pltpu.VMEM((512,), f32),
            pltpu.VMEM((1280,), f32),
            pltpu.SemaphoreType.DMA,
            pltpu.SemaphoreType.DMA,
            pltpu.SemaphoreType.DMA,
            pltpu.SemaphoreType.DMA,
            pltpu.SemaphoreType.DMA,
            pltpu.SemaphoreType.DMA,
        ],
    )
    bxo, lmo = run(
        idx.reshape(B * 128),
        boxes_8.reshape(B * 12800 * 4), boxes_16.reshape(B * 3200 * 4),
        boxes_32.reshape(B * 800 * 4),
        landmarks_8.reshape(B * 12800 * 10),
        landmarks_16.reshape(B * 3200 * 10),
        landmarks_32.reshape(B * 800 * 10))
    det_boxes = jnp.swapaxes(bxo.reshape(B, 4, 128), 1, 2)
    det_landmarks = jnp.swapaxes(lmo.reshape(B, 10, 128), 1, 2)
    return det_boxes, det_landmarks


def kernel(scores_8, boxes_8, landmarks_8, scores_16, boxes_16, landmarks_16,
           scores_32, boxes_32, landmarks_32, anchor_centers, anchor_strides):
    B = scores_8.shape[0]
    scores = jnp.concatenate(
        [scores_8.reshape(B, -1), scores_16.reshape(B, -1),
         scores_32.reshape(B, -1)], axis=1)
    scores = jnp.pad(scores, ((0, 0), (0, _ROWS * 128 - _N)),
                     constant_values=-jnp.inf).reshape(B, _ROWS, 128)

    sig, idx, ndet = pl.pallas_call(
        _topk_body,
        out_shape=[
            jax.ShapeDtypeStruct((B, 128), jnp.float32),
            jax.ShapeDtypeStruct((B, 128), jnp.int32),
            jax.ShapeDtypeStruct((B, 1), jnp.int32),
        ],
        scratch_shapes=[pltpu.VMEM((B, _ROWS, 128), jnp.float32)],
    )(scores)

    det_boxes, det_landmarks = _sc_decode(
        idx, boxes_8, boxes_16, boxes_32,
        landmarks_8, landmarks_16, landmarks_32)
    return (ndet, det_boxes, sig, det_landmarks)


# speedup vs baseline: 1.3136x; 1.3136x over previous
import jax
import jax.numpy as jnp
from jax import lax
from jax.experimental import pallas as pl
from jax.experimental.pallas import tpu as pltpu

_IMG = 640.0
_MAX_DET = 128
_N = 16800
_ROWS = 132


def _topk_body(s8_ref, s16_ref, s32_ref, sig_ref, idx_ref, ndet_ref, S):
    B = 16
    S[:, 0:100, :] = s8_ref[...]
    S[:, 100:125, :] = s16_ref[...]
    S[:, 125:132, :] = s32_ref[...]
    cio = lax.broadcasted_iota(jnp.int32, (B, _ROWS), 1)
    li = lax.broadcasted_iota(jnp.int32, (B, 128), 1)
    M0 = jnp.max(S[...], axis=2)

    def step(j, carry):
        M, sv, iv = carry
        gm = jnp.max(M, axis=1, keepdims=True)
        rvec = jnp.min(jnp.where(M == gm, cio, jnp.int32(100000)),
                       axis=1, keepdims=True)
        rbs = [rvec[b, 0] for b in range(B)]
        R = jnp.concatenate(
            [S[b, pl.ds(rbs[b], 1), :] for b in range(B)], axis=0)
        cvec = jnp.min(jnp.where(R == gm, li, jnp.int32(100000)),
                       axis=1, keepdims=True)
        sv = jnp.where(li == j, gm, sv)
        iv = jnp.where(li == j, rvec * 128 + cvec, iv)
        R2 = jnp.where(li == cvec, -jnp.inf, R)
        for b in range(B):
            S[b, pl.ds(rbs[b], 1), :] = R2[b:b + 1, :]
        nm = jnp.max(R2, axis=1, keepdims=True)
        M = jnp.where(cio == rvec, nm, M)
        return M, sv, iv

    _, sv, iv = lax.fori_loop(
        0, _MAX_DET, step,
        (M0, jnp.full((B, 128), -jnp.inf, jnp.float32),
         jnp.zeros((B, 128), jnp.int32)))
    sig = 1.0 / (1.0 + jnp.exp(-sv))
    sig_ref[...] = sig
    idx_ref[...] = iv
    ndet_ref[...] = jnp.sum(jnp.where(sig > 0.5, 1, 0), axis=1,
                            keepdims=True).astype(jnp.int32)


def kernel(scores_8, boxes_8, landmarks_8, scores_16, boxes_16, landmarks_16,
           scores_32, boxes_32, landmarks_32, anchor_centers, anchor_strides):
    B = scores_8.shape[0]
    del anchor_centers, anchor_strides
    s8 = scores_8.reshape(B, 100, 128)
    s16 = scores_16.reshape(B, 25, 128)
    s32 = jnp.pad(scores_32.reshape(B, 800), ((0, 0), (0, 96)),
                  constant_values=-jnp.inf).reshape(B, 7, 128)

    sig, idx, ndet = pl.pallas_call(
        _topk_body,
        out_shape=[
            jax.ShapeDtypeStruct((B, 128), jnp.float32),
            jax.ShapeDtypeStruct((B, 128), jnp.int32),
            jax.ShapeDtypeStruct((B, 1), jnp.int32),
        ],
        scratch_shapes=[pltpu.VMEM((B, _ROWS, 128), jnp.float32)],
    )(s8, s16, s32)

    bx_all = jnp.concatenate([boxes_8, boxes_16, boxes_32], axis=1)
    lm_all = jnp.concatenate([landmarks_8, landmarks_16, landmarks_32],
                             axis=1)
    bx = jnp.take_along_axis(bx_all, idx[..., None], axis=1)
    lm = jnp.take_along_axis(lm_all, idx[..., None], axis=1)

    in16 = idx >= 12800
    in32 = idx >= 16000
    a = jnp.where(in32, idx - 16000, jnp.where(in16, idx - 12800, idx)) >> 1
    f = jnp.where(in32, 20, jnp.where(in16, 40, 80))
    strd = jnp.where(in32, 32.0, jnp.where(in16, 16.0, 8.0))
    y = a // f
    x = a - y * f
    cx = (x.astype(jnp.float32) + 0.5) * strd
    cy = (y.astype(jnp.float32) + 0.5) * strd
    ctr = jnp.stack([cx, cy], axis=-1)
    st = strd[..., None]

    x1 = ctr[..., 0:1] - bx[..., 0:1] * st
    y1 = ctr[..., 1:2] - bx[..., 1:2] * st
    x2 = ctr[..., 0:1] + bx[..., 2:3] * st
    y2 = ctr[..., 1:2] + bx[..., 3:4] * st
    det_boxes = jnp.concatenate([x1, y1, x2, y2], axis=-1) / _IMG
    lmk = lm.reshape(B, 128, 5, 2)
    det_landmarks = (lmk * st[..., None] + ctr[:, :, None, :]).reshape(
        B, 128, 10) / _IMG
    return (ndet, det_boxes, sig, det_landmarks)
